# baseline (speedup 1.0000x reference)
import functools

import jax
import jax.numpy as jnp
from jax import lax
from jax.experimental import pallas as pl
from jax.experimental.pallas import tpu as pltpu
from jax.experimental.pallas import tpu_sc as plsc

D = 128
DC = 2 * D
NC = 2
NS = 16
NW = NC * NS
CH = 40
G = 10
GB = 2
OB = 5
LANES = 16


def _table_body(x_ref, w_ref, b_ref, c_ref):
    c_ref[...] = (
        jnp.dot(x_ref[...], w_ref[...], preferred_element_type=jnp.float32)
        + b_ref[...]
    ).astype(jnp.bfloat16)


def _build_table(x, wcat, bcat):
    n = x.shape[0]
    blk = 1000
    grid = n // blk
    return pl.pallas_call(
        _table_body,
        grid=(grid,),
        in_specs=[
            pl.BlockSpec((blk, D), lambda i: (i, 0)),
            pl.BlockSpec((D, DC), lambda i: (0, 0)),
            pl.BlockSpec((1, DC), lambda i: (0, 0)),
        ],
        out_specs=pl.BlockSpec((blk, DC), lambda i: (i, 0)),
        out_shape=jax.ShapeDtypeStruct((n, DC), jnp.bfloat16),
    )(x, wcat, bcat)


def _make_edge_kernel(n_edges, n_nodes):
    per_w = n_edges // NW
    n_chunks = per_w // CH
    rows_per_sub = (n_nodes // NS) & ~7
    rows_tail = n_nodes - NS * rows_per_sub

    mesh = plsc.VectorSubcoreMesh(core_axis_name="c", subcore_axis_name="s")

    @functools.partial(
        pl.kernel,
        mesh=mesh,
        out_type=jax.ShapeDtypeStruct((n_edges, D), jnp.float32),
        scratch_types=[pltpu.VMEM_SHARED((n_nodes, D), jnp.int32)]
        + [pltpu.VMEM((2, CH), jnp.int32) for _ in range(G)]
        + [pltpu.VMEM((CH, D), jnp.int32) for _ in range(2 * GB)]
        + [pltpu.VMEM((CH, D), jnp.float32) for _ in range(OB)]
        + [pltpu.SemaphoreType.DMA for _ in range(G + GB + OB)],
    )
    def edge_kernel(c_hbm, idx_hbm, out_hbm, table, *bufs):
        idxb = bufs[0:G]
        srow = bufs[G:G + GB]
        drow = bufs[G + GB:G + 2 * GB]
        orow = bufs[G + 2 * GB:G + 2 * GB + OB]
        isem = bufs[G + 2 * GB + OB:2 * G + 2 * GB + OB]
        gsem = bufs[2 * G + 2 * GB + OB:2 * G + 3 * GB + OB]
        wsem = bufs[2 * G + 3 * GB + OB:2 * G + 3 * GB + 2 * OB]
        sid = lax.axis_index("s")
        wid = sid * NC + lax.axis_index("c")
        base = wid * per_w

        pltpu.sync_copy(c_hbm.at[pl.ds(sid * rows_per_sub, rows_per_sub)],
                        table.at[pl.ds(sid * rows_per_sub, rows_per_sub)])
        if rows_tail:
            @pl.when(sid == 0)
            def _():
                pltpu.sync_copy(
                    c_hbm.at[pl.ds(NS * rows_per_sub, rows_tail)],
                    table.at[pl.ds(NS * rows_per_sub, rows_tail)])
        plsc.subcore_barrier()

        def fire_idx(islot, ci):
            pltpu.async_copy(idx_hbm.at[wid, ci], idxb[islot], isem[islot])

        def wait_idx(islot):
            pltpu.make_async_copy(
                idx_hbm.at[wid, 0], idxb[islot], isem[islot]).wait()

        def fire_gathers(gslot, islot):
            pltpu.async_copy(table.at[idxb[islot].at[0]], srow[gslot],
                             gsem[gslot])
            pltpu.async_copy(table.at[idxb[islot].at[1]], drow[gslot],
                             gsem[gslot])

        def wait_gathers(gslot):
            pltpu.make_async_copy(
                table.at[idxb[0].at[0]], srow[gslot], gsem[gslot]).wait()
            pltpu.make_async_copy(
                table.at[idxb[0].at[1]], drow[gslot], gsem[gslot]).wait()

        def wait_write(oslot):
            pltpu.make_async_copy(
                orow[oslot], out_hbm.at[pl.ds(base, CH)], wsem[oslot]).wait()

        for c in range(G):
            fire_idx(c, c)
        for c in range(GB):
            wait_idx(c)
            fire_gathers(c, c)

        def group_body(gi, carry):
            ci0 = gi * G
            for b in range(G):
                ci = ci0 + b
                gslot = b % GB
                oslot = b % OB
                wait_gathers(gslot)

                @pl.when(ci0 + b + G < n_chunks)
                def _():
                    fire_idx(b, ci + G)

                @pl.when(ci0 + b >= OB)
                def _():
                    wait_write(oslot)

                @plsc.parallel_loop(0, CH, unroll=4)
                def row_body(i):
                    hi_mask = jnp.int32(-65536)
                    for j in range(D // LANES):
                        su = srow[gslot][i, pl.ds(LANES * j, LANES)]
                        du = drow[gslot][i, pl.ds(LANES * j, LANES)]
                        sa = lax.bitcast_convert_type(su << 16, jnp.float32)
                        sb = lax.bitcast_convert_type(su & hi_mask,
                                                      jnp.float32)
                        da = lax.bitcast_convert_type(du << 16, jnp.float32)
                        db = lax.bitcast_convert_type(du & hi_mask,
                                                      jnp.float32)
                        orow[oslot][i, pl.ds(LANES * j, LANES)] = (
                            jnp.maximum(sa + db, 0.0)
                            + jnp.maximum(da + sb, 0.0)
                        )

                pltpu.async_copy(
                    orow[oslot], out_hbm.at[pl.ds(base + ci * CH, CH)],
                    wsem[oslot])

                @pl.when(ci0 + b + GB < n_chunks)
                def _():
                    wait_idx((b + GB) % G)
                    fire_gathers(gslot, (b + GB) % G)

            return carry

        lax.fori_loop(0, n_chunks // G, group_body, 0)

        for oslot in range(OB):
            wait_write(oslot)

    return edge_kernel


def kernel(x, edge_index, W, b):
    n_edges = edge_index.shape[1]
    per_w = n_edges // NW
    n_chunks = per_w // CH
    wcat = jnp.concatenate([W[:D], W[D:]], axis=1)
    bcat = jnp.concatenate([jnp.zeros_like(b), b]).reshape(1, DC)
    perm = jnp.stack([jnp.arange(D), jnp.arange(D) + D], axis=1).reshape(-1)
    c_bf = _build_table(x, wcat[:, perm], bcat[:, perm])
    c = lax.bitcast_convert_type(
        c_bf.reshape(x.shape[0], D, 2), jnp.int32)
    idx = edge_index.astype(jnp.int32).reshape(2, NW, n_chunks, CH)
    idx = jnp.transpose(idx, (1, 2, 0, 3))
    return _make_edge_kernel(n_edges, x.shape[0])(c, idx)


# speedup vs baseline: 1.2984x; 1.2984x over previous
import functools

import jax
import jax.numpy as jnp
from jax import lax
from jax.experimental import pallas as pl
from jax.experimental.pallas import tpu as pltpu
from jax.experimental.pallas import tpu_sc as plsc

D = 128
DC = 2 * D
NC = 2
NS = 16
NW = NC * NS
CH = 40
G = 10
GB = 2
OB = 5
LANES = 16


def _table_body(x_ref, w_ref, b_ref, c_ref):
    c_ref[...] = (
        jnp.dot(x_ref[...], w_ref[...], preferred_element_type=jnp.float32)
        + b_ref[...]
    ).astype(jnp.bfloat16)


def _build_table(x, wcat, bcat):
    n = x.shape[0]
    blk = 1000
    grid = n // blk
    return pl.pallas_call(
        _table_body,
        grid=(grid,),
        in_specs=[
            pl.BlockSpec((blk, D), lambda i: (i, 0)),
            pl.BlockSpec((D, DC), lambda i: (0, 0)),
            pl.BlockSpec((1, DC), lambda i: (0, 0)),
        ],
        out_specs=pl.BlockSpec((blk, DC), lambda i: (i, 0)),
        out_shape=jax.ShapeDtypeStruct((n, DC), jnp.bfloat16),
    )(x, wcat, bcat)


def _make_edge_kernel(n_edges, n_nodes):
    per_w = n_edges // NW
    n_chunks = per_w // CH
    rows_per_sub = (n_nodes // NS) & ~7
    rows_tail = n_nodes - NS * rows_per_sub

    mesh = plsc.VectorSubcoreMesh(core_axis_name="c", subcore_axis_name="s")

    @functools.partial(
        pl.kernel,
        mesh=mesh,
        out_type=jax.ShapeDtypeStruct((n_edges, D), jnp.float32),
        scratch_types=[pltpu.VMEM_SHARED((n_nodes, D), jnp.int32)]
        + [pltpu.VMEM((2, CH), jnp.int32) for _ in range(G)]
        + [pltpu.VMEM((CH, D), jnp.int32) for _ in range(2 * GB)]
        + [pltpu.VMEM((CH, D), jnp.float32) for _ in range(OB)]
        + [pltpu.SemaphoreType.DMA for _ in range(G + GB + OB)],
    )
    def edge_kernel(c_hbm, idx_hbm, out_hbm, table, *bufs):
        idxb = bufs[0:G]
        srow = bufs[G:G + GB]
        drow = bufs[G + GB:G + 2 * GB]
        orow = bufs[G + 2 * GB:G + 2 * GB + OB]
        isem = bufs[G + 2 * GB + OB:2 * G + 2 * GB + OB]
        gsem = bufs[2 * G + 2 * GB + OB:2 * G + 3 * GB + OB]
        wsem = bufs[2 * G + 3 * GB + OB:2 * G + 3 * GB + 2 * OB]
        sid = lax.axis_index("s")
        wid = sid * NC + lax.axis_index("c")
        base = wid * per_w

        pltpu.sync_copy(c_hbm.at[pl.ds(sid * rows_per_sub, rows_per_sub)],
                        table.at[pl.ds(sid * rows_per_sub, rows_per_sub)])
        if rows_tail:
            @pl.when(sid == 0)
            def _():
                pltpu.sync_copy(
                    c_hbm.at[pl.ds(NS * rows_per_sub, rows_tail)],
                    table.at[pl.ds(NS * rows_per_sub, rows_tail)])
        plsc.subcore_barrier()

        def fire_idx(islot, ci):
            pltpu.async_copy(idx_hbm.at[wid, ci], idxb[islot], isem[islot])

        def wait_idx(islot):
            pltpu.make_async_copy(
                idx_hbm.at[wid, 0], idxb[islot], isem[islot]).wait()

        def fire_gathers(gslot, islot):
            pltpu.async_copy(table.at[idxb[islot].at[0]], srow[gslot],
                             gsem[gslot])
            pltpu.async_copy(table.at[idxb[islot].at[1]], drow[gslot],
                             gsem[gslot])

        def wait_gathers(gslot):
            pltpu.make_async_copy(
                table.at[idxb[0].at[0]], srow[gslot], gsem[gslot]).wait()
            pltpu.make_async_copy(
                table.at[idxb[0].at[1]], drow[gslot], gsem[gslot]).wait()

        def wait_write(oslot):
            pltpu.make_async_copy(
                orow[oslot], out_hbm.at[pl.ds(base, CH)], wsem[oslot]).wait()

        for c in range(G):
            fire_idx(c, c)
        for c in range(GB):
            wait_idx(c)
            fire_gathers(c, c)

        def group_body(gi, carry):
            ci0 = gi * G
            for b in range(G):
                ci = ci0 + b
                gslot = b % GB
                oslot = b % OB
                wait_gathers(gslot)

                @pl.when(ci0 + b + G < n_chunks)
                def _():
                    fire_idx(b, ci + G)

                @pl.when(ci0 + b >= OB)
                def _():
                    wait_write(oslot)

                @plsc.parallel_loop(0, CH, unroll=1)
                def row_body(i):
                    hi_mask = jnp.int32(-65536)
                    for j in range(D // LANES):
                        su = srow[gslot][i, pl.ds(LANES * j, LANES)]
                        du = drow[gslot][i, pl.ds(LANES * j, LANES)]
                        sa = lax.bitcast_convert_type(su << 16, jnp.float32)
                        sb = lax.bitcast_convert_type(su & hi_mask,
                                                      jnp.float32)
                        da = lax.bitcast_convert_type(du << 16, jnp.float32)
                        db = lax.bitcast_convert_type(du & hi_mask,
                                                      jnp.float32)
                        orow[oslot][i, pl.ds(LANES * j, LANES)] = (
                            jnp.maximum(sa + db, 0.0)
                            + jnp.maximum(da + sb, 0.0)
                        )

                pltpu.async_copy(
                    orow[oslot], out_hbm.at[pl.ds(base + ci * CH, CH)],
                    wsem[oslot])

                @pl.when(ci0 + b + GB < n_chunks)
                def _():
                    wait_idx((b + GB) % G)
                    fire_gathers(gslot, (b + GB) % G)

            return carry

        lax.fori_loop(0, n_chunks // G, group_body, 0)

        for oslot in range(OB):
            wait_write(oslot)

    return edge_kernel


def kernel(x, edge_index, W, b):
    n_edges = edge_index.shape[1]
    per_w = n_edges // NW
    n_chunks = per_w // CH
    wcat = jnp.concatenate([W[:D], W[D:]], axis=1)
    bcat = jnp.concatenate([jnp.zeros_like(b), b]).reshape(1, DC)
    perm = jnp.stack([jnp.arange(D), jnp.arange(D) + D], axis=1).reshape(-1)
    c_bf = _build_table(x, wcat[:, perm], bcat[:, perm])
    c = lax.bitcast_convert_type(
        c_bf.reshape(x.shape[0], D, 2), jnp.int32)
    idx = edge_index.astype(jnp.int32).reshape(2, NW, n_chunks, CH)
    idx = jnp.transpose(idx, (1, 2, 0, 3))
    return _make_edge_kernel(n_edges, x.shape[0])(c, idx)


# speedup vs baseline: 1.3317x; 1.0256x over previous
import functools

import jax
import jax.numpy as jnp
from jax import lax
from jax.experimental import pallas as pl
from jax.experimental.pallas import tpu as pltpu
from jax.experimental.pallas import tpu_sc as plsc

D = 128
DC = 2 * D
NC = 2
NS = 16
NW = NC * NS
CH = 40
G = 10
GB = 2
OB = 5
LANES = 16


def _table_body(x_ref, w_ref, b_ref, c_ref):
    c_ref[...] = (
        jnp.dot(x_ref[...], w_ref[...], preferred_element_type=jnp.float32)
        + b_ref[...]
    ).astype(jnp.bfloat16)


def _build_table(x, wcat, bcat):
    n = x.shape[0]
    blk = 1000
    grid = n // blk
    return pl.pallas_call(
        _table_body,
        grid=(grid,),
        in_specs=[
            pl.BlockSpec((blk, D), lambda i: (i, 0)),
            pl.BlockSpec((D, DC), lambda i: (0, 0)),
            pl.BlockSpec((1, DC), lambda i: (0, 0)),
        ],
        out_specs=pl.BlockSpec((blk, DC), lambda i: (i, 0)),
        out_shape=jax.ShapeDtypeStruct((n, DC), jnp.bfloat16),
    )(x, wcat, bcat)


def _make_edge_kernel(n_edges, n_nodes):
    per_w = n_edges // NW
    n_chunks = per_w // CH
    rows_per_sub = (n_nodes // NS) & ~7
    rows_tail = n_nodes - NS * rows_per_sub

    mesh = plsc.VectorSubcoreMesh(core_axis_name="c", subcore_axis_name="s")

    @functools.partial(
        pl.kernel,
        mesh=mesh,
        out_type=jax.ShapeDtypeStruct((n_edges, D), jnp.float32),
        scratch_types=[pltpu.VMEM_SHARED((n_nodes, D), jnp.int32)]
        + [pltpu.VMEM((2 * CH,), jnp.int32) for _ in range(G)]
        + [pltpu.VMEM((2 * CH, D), jnp.int32) for _ in range(GB)]
        + [pltpu.VMEM((CH, D), jnp.float32) for _ in range(OB)]
        + [pltpu.SemaphoreType.DMA for _ in range(G + GB + OB)],
    )
    def edge_kernel(c_hbm, idx_hbm, out_hbm, table, *bufs):
        idxb = bufs[0:G]
        grow = bufs[G:G + GB]
        orow = bufs[G + GB:G + GB + OB]
        isem = bufs[G + GB + OB:2 * G + GB + OB]
        gsem = bufs[2 * G + GB + OB:2 * G + 2 * GB + OB]
        wsem = bufs[2 * G + 2 * GB + OB:2 * G + 2 * GB + 2 * OB]
        sid = lax.axis_index("s")
        wid = sid * NC + lax.axis_index("c")
        base = wid * per_w

        pltpu.sync_copy(c_hbm.at[pl.ds(sid * rows_per_sub, rows_per_sub)],
                        table.at[pl.ds(sid * rows_per_sub, rows_per_sub)])
        if rows_tail:
            @pl.when(sid == 0)
            def _():
                pltpu.sync_copy(
                    c_hbm.at[pl.ds(NS * rows_per_sub, rows_tail)],
                    table.at[pl.ds(NS * rows_per_sub, rows_tail)])
        plsc.subcore_barrier()

        def fire_idx(islot, ci):
            pltpu.async_copy(idx_hbm.at[wid, ci], idxb[islot], isem[islot])

        def wait_idx(islot):
            pltpu.make_async_copy(
                idx_hbm.at[wid, 0], idxb[islot], isem[islot]).wait()

        def fire_gathers(gslot, islot):
            pltpu.async_copy(table.at[idxb[islot]], grow[gslot], gsem[gslot])

        def wait_gathers(gslot):
            pltpu.make_async_copy(
                table.at[idxb[0]], grow[gslot], gsem[gslot]).wait()

        def wait_write(oslot):
            pltpu.make_async_copy(
                orow[oslot], out_hbm.at[pl.ds(base, CH)], wsem[oslot]).wait()

        for c in range(G):
            fire_idx(c, c)
        for c in range(GB):
            wait_idx(c)
            fire_gathers(c, c)

        def group_body(gi, carry):
            ci0 = gi * G
            for b in range(G):
                ci = ci0 + b
                gslot = b % GB
                oslot = b % OB
                wait_gathers(gslot)

                @pl.when(ci0 + b + G < n_chunks)
                def _():
                    fire_idx(b, ci + G)

                @pl.when(ci0 + b >= OB)
                def _():
                    wait_write(oslot)

                @plsc.parallel_loop(0, CH, unroll=1)
                def row_body(i):
                    hi_mask = jnp.int32(-65536)
                    for j in range(D // LANES):
                        su = grow[gslot][i, pl.ds(LANES * j, LANES)]
                        du = grow[gslot][CH + i, pl.ds(LANES * j, LANES)]
                        sa = lax.bitcast_convert_type(su << 16, jnp.float32)
                        sb = lax.bitcast_convert_type(su & hi_mask,
                                                      jnp.float32)
                        da = lax.bitcast_convert_type(du << 16, jnp.float32)
                        db = lax.bitcast_convert_type(du & hi_mask,
                                                      jnp.float32)
                        orow[oslot][i, pl.ds(LANES * j, LANES)] = (
                            jnp.maximum(sa + db, 0.0)
                            + jnp.maximum(da + sb, 0.0)
                        )

                pltpu.async_copy(
                    orow[oslot], out_hbm.at[pl.ds(base + ci * CH, CH)],
                    wsem[oslot])

                @pl.when(ci0 + b + GB < n_chunks)
                def _():
                    wait_idx((b + GB) % G)
                    fire_gathers(gslot, (b + GB) % G)

            return carry

        lax.fori_loop(0, n_chunks // G, group_body, 0)

        for oslot in range(OB):
            wait_write(oslot)

    return edge_kernel


def kernel(x, edge_index, W, b):
    n_edges = edge_index.shape[1]
    per_w = n_edges // NW
    n_chunks = per_w // CH
    wcat = jnp.concatenate([W[:D], W[D:]], axis=1)
    bcat = jnp.concatenate([jnp.zeros_like(b), b]).reshape(1, DC)
    perm = jnp.stack([jnp.arange(D), jnp.arange(D) + D], axis=1).reshape(-1)
    c_bf = _build_table(x, wcat[:, perm], bcat[:, perm])
    c = lax.bitcast_convert_type(
        c_bf.reshape(x.shape[0], D, 2), jnp.int32)
    idx = edge_index.astype(jnp.int32).reshape(2, NW, n_chunks, CH)
    idx = jnp.transpose(idx, (1, 2, 0, 3))
    idx = idx.reshape(NW, n_chunks, 2 * CH)
    return _make_edge_kernel(n_edges, x.shape[0])(c, idx)


# speedup vs baseline: 2.0058x; 1.5062x over previous
import functools

import jax
import jax.numpy as jnp
from jax import lax
from jax.experimental import pallas as pl
from jax.experimental.pallas import tpu as pltpu
from jax.experimental.pallas import tpu_sc as plsc

D = 128
DC = 2 * D
NC = 2
NS = 16
NW = NC * NS
CH = 40
G = 10
GB = 2
OB = 5
LANES = 16


def _table_body(x_ref, w_ref, b_ref, c_ref):
    xv = x_ref[...]
    a = jnp.dot(xv, w_ref[0:D, :], preferred_element_type=jnp.float32)
    bv = (jnp.dot(xv, w_ref[D:2 * D, :], preferred_element_type=jnp.float32)
          + b_ref[...])
    au = lax.bitcast_convert_type(
        a.astype(jnp.bfloat16), jnp.uint16).astype(jnp.uint32)
    bu = lax.bitcast_convert_type(
        bv.astype(jnp.bfloat16), jnp.uint16).astype(jnp.uint32)
    c_ref[...] = lax.bitcast_convert_type(au | (bu << 16), jnp.int32)


def _build_table(x, w, brow):
    n = x.shape[0]
    blk = 1000
    grid = n // blk
    return pl.pallas_call(
        _table_body,
        grid=(grid,),
        in_specs=[
            pl.BlockSpec((blk, D), lambda i: (i, 0)),
            pl.BlockSpec((DC, D), lambda i: (0, 0)),
            pl.BlockSpec((1, D), lambda i: (0, 0)),
        ],
        out_specs=pl.BlockSpec((blk, D), lambda i: (i, 0)),
        out_shape=jax.ShapeDtypeStruct((n, D), jnp.int32),
    )(x, w, brow)


def _make_edge_kernel(n_edges, n_nodes):
    per_w = n_edges // NW
    n_chunks = per_w // CH
    rows_per_sub = (n_nodes // NS) & ~7
    rows_tail = n_nodes - NS * rows_per_sub

    mesh = plsc.VectorSubcoreMesh(core_axis_name="c", subcore_axis_name="s")

    @functools.partial(
        pl.kernel,
        mesh=mesh,
        out_type=jax.ShapeDtypeStruct((n_edges, D), jnp.float32),
        scratch_types=[pltpu.VMEM_SHARED((n_nodes, D), jnp.int32)]
        + [pltpu.VMEM((2 * CH,), jnp.int32) for _ in range(G)]
        + [pltpu.VMEM((2 * CH, D), jnp.int32) for _ in range(GB)]
        + [pltpu.VMEM((CH, D), jnp.float32) for _ in range(OB)]
        + [pltpu.SemaphoreType.DMA for _ in range(G + GB + OB)],
    )
    def edge_kernel(c_hbm, idx_hbm, out_hbm, table, *bufs):
        idxb = bufs[0:G]
        grow = bufs[G:G + GB]
        orow = bufs[G + GB:G + GB + OB]
        isem = bufs[G + GB + OB:2 * G + GB + OB]
        gsem = bufs[2 * G + GB + OB:2 * G + 2 * GB + OB]
        wsem = bufs[2 * G + 2 * GB + OB:2 * G + 2 * GB + 2 * OB]
        sid = lax.axis_index("s")
        wid = sid * NC + lax.axis_index("c")
        base = wid * per_w

        pltpu.sync_copy(c_hbm.at[pl.ds(sid * rows_per_sub, rows_per_sub)],
                        table.at[pl.ds(sid * rows_per_sub, rows_per_sub)])
        if rows_tail:
            @pl.when(sid == 0)
            def _():
                pltpu.sync_copy(
                    c_hbm.at[pl.ds(NS * rows_per_sub, rows_tail)],
                    table.at[pl.ds(NS * rows_per_sub, rows_tail)])
        plsc.subcore_barrier()

        def fire_idx(islot, ci):
            pltpu.async_copy(idx_hbm.at[wid, ci], idxb[islot], isem[islot])

        def wait_idx(islot):
            pltpu.make_async_copy(
                idx_hbm.at[wid, 0], idxb[islot], isem[islot]).wait()

        def fire_gathers(gslot, islot):
            pltpu.async_copy(table.at[idxb[islot]], grow[gslot], gsem[gslot])

        def wait_gathers(gslot):
            pltpu.make_async_copy(
                table.at[idxb[0]], grow[gslot], gsem[gslot]).wait()

        def wait_write(oslot):
            pltpu.make_async_copy(
                orow[oslot], out_hbm.at[pl.ds(base, CH)], wsem[oslot]).wait()

        for c in range(G):
            fire_idx(c, c)
        for c in range(GB):
            wait_idx(c)
            fire_gathers(c, c)

        def group_body(gi, carry):
            ci0 = gi * G
            for b in range(G):
                ci = ci0 + b
                gslot = b % GB
                oslot = b % OB
                wait_gathers(gslot)

                @pl.when(ci0 + b + G < n_chunks)
                def _():
                    fire_idx(b, ci + G)

                @pl.when(ci0 + b >= OB)
                def _():
                    wait_write(oslot)

                @plsc.parallel_loop(0, CH, unroll=1)
                def row_body(i):
                    hi_mask = jnp.int32(-65536)
                    for j in range(D // LANES):
                        su = grow[gslot][i, pl.ds(LANES * j, LANES)]
                        du = grow[gslot][CH + i, pl.ds(LANES * j, LANES)]
                        sa = lax.bitcast_convert_type(su << 16, jnp.float32)
                        sb = lax.bitcast_convert_type(su & hi_mask,
                                                      jnp.float32)
                        da = lax.bitcast_convert_type(du << 16, jnp.float32)
                        db = lax.bitcast_convert_type(du & hi_mask,
                                                      jnp.float32)
                        orow[oslot][i, pl.ds(LANES * j, LANES)] = (
                            jnp.maximum(sa + db, 0.0)
                            + jnp.maximum(da + sb, 0.0)
                        )

                pltpu.async_copy(
                    orow[oslot], out_hbm.at[pl.ds(base + ci * CH, CH)],
                    wsem[oslot])

                @pl.when(ci0 + b + GB < n_chunks)
                def _():
                    wait_idx((b + GB) % G)
                    fire_gathers(gslot, (b + GB) % G)

            return carry

        lax.fori_loop(0, n_chunks // G, group_body, 0)

        for oslot in range(OB):
            wait_write(oslot)

    return edge_kernel


def kernel(x, edge_index, W, b):
    n_edges = edge_index.shape[1]
    per_w = n_edges // NW
    n_chunks = per_w // CH
    c = _build_table(x, W, b.reshape(1, D))
    idx = edge_index.astype(jnp.int32).reshape(2, NW, n_chunks, CH)
    idx = jnp.transpose(idx, (1, 2, 0, 3)).reshape(NW, n_chunks, 2 * CH)
    return _make_edge_kernel(n_edges, x.shape[0])(c, idx)
